# baseline (speedup 1.0000x reference)
import functools

import jax
import jax.numpy as jnp
from jax import lax
from jax.experimental import pallas as pl
from jax.experimental.pallas import tpu as pltpu
from jax.experimental.pallas import tpu_sc as plsc

N_TOTAL = 64 * 32768
NC = 2
NS = 16
L = 16
NW = NC * NS
PER_W = N_TOTAL // NW
CH = 16384
NCH = PER_W // CH

_mesh = plsc.VectorSubcoreMesh(core_axis_name="c", subcore_axis_name="s")


@functools.partial(
    pl.kernel,
    mesh=_mesh,
    out_type=jax.ShapeDtypeStruct((NW, 3 * L), jnp.float32),
    scratch_types=[
        pltpu.VMEM((CH,), jnp.float32),
        pltpu.VMEM((CH,), jnp.float32),
        pltpu.VMEM((3 * L,), jnp.float32),
    ],
)
def _partial_sums(x_hbm, t_hbm, out_hbm, xv, tv, outv):
    wid = lax.axis_index("s") * NC + lax.axis_index("c")
    base = wid * PER_W
    zero = jnp.zeros((L,), jnp.float32)

    def chunk_body(c, carry):
        off = base + c * CH
        pltpu.sync_copy(x_hbm.at[pl.ds(off, CH)], xv)
        pltpu.sync_copy(t_hbm.at[pl.ds(off, CH)], tv)

        def vec_body(j, acc):
            a_bt, a_sq, a_ct = acc
            x = xv[pl.ds(j * L, L)]
            t = tv[pl.ds(j * L, L)]
            d = x - t
            sq = d * d
            return (a_bt + sq * t, a_sq + sq, a_ct + t)

        return lax.fori_loop(0, CH // L, vec_body, carry)

    a_bt, a_sq, a_ct = lax.fori_loop(0, NCH, chunk_body, (zero, zero, zero))
    outv[pl.ds(0, L)] = a_bt
    outv[pl.ds(L, L)] = a_sq
    outv[pl.ds(2 * L, L)] = a_ct
    pltpu.sync_copy(outv, out_hbm.at[wid])


def _finish(p_ref, o_ref):
    p = p_ref[...]
    bt = jnp.sum(p[:, 0:L])
    tot = jnp.sum(p[:, L:2 * L])
    ct = jnp.sum(p[:, 2 * L:3 * L])
    beat_count = jnp.maximum(ct, 1.0)
    no_beat_count = jnp.maximum(jnp.float32(N_TOTAL) - ct, 1.0)
    beat_loss = bt / beat_count
    no_beat_loss = (tot - bt) / no_beat_count
    o_ref[0] = no_beat_loss + beat_loss
    o_ref[1] = beat_loss
    o_ref[2] = no_beat_loss


_finish_call = pl.pallas_call(
    _finish,
    out_shape=jax.ShapeDtypeStruct((3,), jnp.float32),
    out_specs=pl.BlockSpec(memory_space=pltpu.SMEM),
)


def kernel(input, target):
    x = input.reshape(N_TOTAL)
    t = target.reshape(N_TOTAL)
    partials = _partial_sums(x, t)
    out = _finish_call(partials)
    return (out[0], out[1], out[2])


# speedup vs baseline: 1.4147x; 1.4147x over previous
import functools

import jax
import jax.numpy as jnp
from jax import lax
from jax.experimental import pallas as pl
from jax.experimental.pallas import tpu as pltpu
from jax.experimental.pallas import tpu_sc as plsc

N_TOTAL = 64 * 32768
NC = 2
NS = 16
L = 16
NW = NC * NS
PER_W = N_TOTAL // NW
CH = 16384
NCH_ROW = 32768 // CH
NCH = PER_W // CH

_mesh = plsc.VectorSubcoreMesh(core_axis_name="c", subcore_axis_name="s")


@functools.partial(
    pl.kernel,
    mesh=_mesh,
    out_type=jax.ShapeDtypeStruct((NW, 3 * L), jnp.float32),
    scratch_types=[
        pltpu.VMEM((CH,), jnp.float32),
        pltpu.VMEM((CH,), jnp.float32),
        pltpu.VMEM((3 * L,), jnp.float32),
    ],
)
def _partial_sums(x_hbm, t_hbm, out_hbm, xv, tv, outv):
    wid = lax.axis_index("s") * NC + lax.axis_index("c")
    row0 = wid * 2
    zero = jnp.zeros((L,), jnp.float32)

    def chunk_body(c, carry):
        row = row0 + c // NCH_ROW
        off = (c % NCH_ROW) * CH
        pltpu.sync_copy(x_hbm.at[row, pl.ds(off, CH)], xv)
        pltpu.sync_copy(t_hbm.at[row, pl.ds(off, CH)], tv)

        def vec_body(j, acc):
            a_bt, a_sq, a_ct = acc
            x = xv[pl.ds(j * L, L)]
            t = tv[pl.ds(j * L, L)]
            d = x - t
            sq = d * d
            return (a_bt + sq * t, a_sq + sq, a_ct + t)

        return lax.fori_loop(0, CH // L, vec_body, carry)

    a_bt, a_sq, a_ct = lax.fori_loop(0, NCH, chunk_body, (zero, zero, zero))
    outv[pl.ds(0, L)] = a_bt
    outv[pl.ds(L, L)] = a_sq
    outv[pl.ds(2 * L, L)] = a_ct
    pltpu.sync_copy(outv, out_hbm.at[wid])


def _finish(p_ref, o_ref):
    p = p_ref[...]
    bt = jnp.sum(p[:, 0:L])
    tot = jnp.sum(p[:, L:2 * L])
    ct = jnp.sum(p[:, 2 * L:3 * L])
    beat_count = jnp.maximum(ct, 1.0)
    no_beat_count = jnp.maximum(jnp.float32(N_TOTAL) - ct, 1.0)
    beat_loss = bt / beat_count
    no_beat_loss = (tot - bt) / no_beat_count
    o_ref[0] = no_beat_loss + beat_loss
    o_ref[1] = beat_loss
    o_ref[2] = no_beat_loss


_finish_call = pl.pallas_call(
    _finish,
    out_shape=jax.ShapeDtypeStruct((3,), jnp.float32),
    out_specs=pl.BlockSpec(memory_space=pltpu.SMEM),
)


def kernel(input, target):
    partials = _partial_sums(input, target)
    out = _finish_call(partials)
    return (out[0], out[1], out[2])


# speedup vs baseline: 2.1561x; 1.5240x over previous
import functools

import jax
import jax.numpy as jnp
from jax import lax
from jax.experimental import pallas as pl
from jax.experimental.pallas import tpu as pltpu
from jax.experimental.pallas import tpu_sc as plsc

N_TOTAL = 64 * 32768
NC = 2
NS = 16
L = 16
NW = NC * NS
PER_W = N_TOTAL // NW
CH = 16384
NCH_ROW = 32768 // CH
NCH = PER_W // CH

_mesh = plsc.VectorSubcoreMesh(core_axis_name="c", subcore_axis_name="s")


@functools.partial(
    pl.kernel,
    mesh=_mesh,
    out_type=jax.ShapeDtypeStruct((NW, 3 * L), jnp.float32),
    scratch_types=[
        pltpu.VMEM((2, CH), jnp.float32),
        pltpu.VMEM((2, CH), jnp.float32),
        pltpu.VMEM((3 * L,), jnp.float32),
        pltpu.SemaphoreType.DMA,
        pltpu.SemaphoreType.DMA,
        pltpu.SemaphoreType.DMA,
        pltpu.SemaphoreType.DMA,
    ],
)
def _partial_sums(x_hbm, t_hbm, out_hbm, xv, tv, outv, sx0, sx1, st0, st1):
    wid = lax.axis_index("s") * NC + lax.axis_index("c")
    row0 = wid * 2
    zero = jnp.zeros((L,), jnp.float32)
    xsems = (sx0, sx1)
    tsems = (st0, st1)

    chunks = [(c // NCH_ROW, (c % NCH_ROW) * CH) for c in range(NCH)]

    def start(i):
        r, off = chunks[i]
        b = i % 2
        hx = pltpu.async_copy(x_hbm.at[row0 + r, pl.ds(off, CH)],
                              xv.at[b], xsems[b])
        ht = pltpu.async_copy(t_hbm.at[row0 + r, pl.ds(off, CH)],
                              tv.at[b], tsems[b])
        return (hx, ht)

    U = 8
    accs = [zero] * (3 * U)
    handles = {0: start(0)}
    for i in range(NCH):
        if i + 1 < NCH:
            handles[i + 1] = start(i + 1)
        hx, ht = handles.pop(i)
        hx.wait()
        ht.wait()
        b = i % 2

        def vec_body(j, acc, _b=b):
            acc = list(acc)
            base = j * (U * L)
            for k in range(U):
                x = xv[_b, pl.ds(base + k * L, L)]
                t = tv[_b, pl.ds(base + k * L, L)]
                d = x - t
                sq = d * d
                acc[k] = acc[k] + sq * t
                acc[U + k] = acc[U + k] + sq
                acc[2 * U + k] = acc[2 * U + k] + t
            return tuple(acc)

        accs = lax.fori_loop(0, CH // (U * L), vec_body, tuple(accs))

    a_bt = functools.reduce(lambda a, b: a + b, accs[0:U])
    a_sq = functools.reduce(lambda a, b: a + b, accs[U:2 * U])
    a_ct = functools.reduce(lambda a, b: a + b, accs[2 * U:3 * U])
    outv[pl.ds(0, L)] = a_bt
    outv[pl.ds(L, L)] = a_sq
    outv[pl.ds(2 * L, L)] = a_ct
    pltpu.sync_copy(outv, out_hbm.at[wid])


def _finish(p_ref, o_ref):
    p = p_ref[...]
    bt = jnp.sum(p[:, 0:L])
    tot = jnp.sum(p[:, L:2 * L])
    ct = jnp.sum(p[:, 2 * L:3 * L])
    beat_count = jnp.maximum(ct, 1.0)
    no_beat_count = jnp.maximum(jnp.float32(N_TOTAL) - ct, 1.0)
    beat_loss = bt / beat_count
    no_beat_loss = (tot - bt) / no_beat_count
    o_ref[0] = no_beat_loss + beat_loss
    o_ref[1] = beat_loss
    o_ref[2] = no_beat_loss


_finish_call = pl.pallas_call(
    _finish,
    out_shape=jax.ShapeDtypeStruct((3,), jnp.float32),
    out_specs=pl.BlockSpec(memory_space=pltpu.SMEM),
)


def kernel(input, target):
    partials = _partial_sums(input, target)
    out = _finish_call(partials)
    return (out[0], out[1], out[2])
